# baseline (speedup 1.0000x reference)
import functools
import math

import jax
import jax.numpy as jnp
from jax import lax
from jax.experimental import pallas as pl
from jax.experimental.pallas import tpu as pltpu
from jax.experimental.pallas import tpu_sc as plsc

D_MODEL = 64
LANES = 16
NUM_CORES = 2
NUM_SUBCORES = 16
NUM_WORKERS = NUM_CORES * NUM_SUBCORES
CHUNK = 128
NBUF = 4
SCALE = math.sqrt(D_MODEL)


def _emb_body(idx_hbm, lut_hbm, out_hbm, idx_v, gbuf, sbuf, gsem, ssem):
    nchunks = idx_hbm.shape[0] // NUM_WORKERS
    wid = lax.axis_index("s") * NUM_CORES + lax.axis_index("c")
    chunk0 = wid * nchunks
    row0 = chunk0 * CHUNK

    pltpu.sync_copy(idx_hbm.at[pl.ds(chunk0, nchunks)], idx_v)

    for b in range(NBUF):
        pltpu.async_copy(lut_hbm.at[idx_v.at[b]], gbuf.at[b], gsem.at[b])

    nouter = nchunks // NBUF

    def outer(o, _):
        for b in range(NBUF):
            g = o * NBUF + b
            pltpu.make_async_copy(
                lut_hbm.at[idx_v.at[b]], gbuf.at[b], gsem.at[b]).wait()

            @pl.when(o > 0)
            def _wait_scatter():
                pltpu.make_async_copy(
                    sbuf.at[b], out_hbm.at[pl.ds(row0, CHUNK)],
                    ssem.at[b]).wait()

            def scale_row(i, _):
                for j in range(D_MODEL // LANES):
                    sbuf[b, i, pl.ds(j * LANES, LANES)] = (
                        gbuf[b, i, pl.ds(j * LANES, LANES)] * SCALE)
                return 0

            lax.fori_loop(0, CHUNK, scale_row, 0, unroll=8)

            @pl.when(g + NBUF < nchunks)
            def _next_gather():
                pltpu.async_copy(
                    lut_hbm.at[idx_v.at[g + NBUF]], gbuf.at[b], gsem.at[b])

            pltpu.async_copy(
                sbuf.at[b], out_hbm.at[pl.ds(row0 + g * CHUNK, CHUNK)],
                ssem.at[b])
        return 0

    lax.fori_loop(0, nouter, outer, 0)

    for b in range(NBUF):
        pltpu.make_async_copy(
            sbuf.at[b], out_hbm.at[pl.ds(row0, CHUNK)], ssem.at[b]).wait()


def kernel(x, lut):
    n_rows, n_cols = x.shape
    total = n_rows * n_cols
    idx = x.reshape(total // CHUNK, CHUNK).astype(jnp.int32)

    emb_flat = pl.kernel(
        _emb_body,
        out_type=jax.ShapeDtypeStruct((total, D_MODEL), jnp.float32),
        mesh=plsc.VectorSubcoreMesh(core_axis_name="c", subcore_axis_name="s"),
        compiler_params=pltpu.CompilerParams(use_tc_tiling_on_sc=False),
        scratch_types=[
            pltpu.VMEM((total // CHUNK // NUM_WORKERS, CHUNK), jnp.int32),
            pltpu.VMEM((NBUF, CHUNK, D_MODEL), jnp.float32),
            pltpu.VMEM((NBUF, CHUNK, D_MODEL), jnp.float32),
            pltpu.SemaphoreType.DMA((NBUF,)),
            pltpu.SemaphoreType.DMA((NBUF,)),
        ],
    )(idx, lut)

    return emb_flat.reshape(n_rows, n_cols, D_MODEL)


# speedup vs baseline: 1.0009x; 1.0009x over previous
import functools
import math

import jax
import jax.numpy as jnp
from jax import lax
from jax.experimental import pallas as pl
from jax.experimental.pallas import tpu as pltpu
from jax.experimental.pallas import tpu_sc as plsc

D_MODEL = 64
LANES = 16
NUM_CORES = 2
NUM_SUBCORES = 16
NUM_WORKERS = NUM_CORES * NUM_SUBCORES
CHUNK = 128
NBUF = 4
SCALE = math.sqrt(D_MODEL)


def _emb_body(idx_hbm, lut_hbm, out_hbm, idx_v, gbuf, sbuf, gsem, ssem):
    nchunks = idx_hbm.shape[0] // NUM_WORKERS
    wid = lax.axis_index("s") * NUM_CORES + lax.axis_index("c")
    chunk0 = wid * nchunks
    row0 = chunk0 * CHUNK

    pltpu.sync_copy(idx_hbm.at[pl.ds(chunk0, nchunks)], idx_v)

    for b in range(NBUF):
        pltpu.async_copy(lut_hbm.at[idx_v.at[b]], gbuf.at[b], gsem.at[b])

    nouter = nchunks // NBUF

    def outer(o, _):
        for b in range(NBUF):
            g = o * NBUF + b
            pltpu.make_async_copy(
                lut_hbm.at[idx_v.at[b]], gbuf.at[b], gsem.at[b]).wait()

            @pl.when(o > 0)
            def _wait_scatter():
                pltpu.make_async_copy(
                    sbuf.at[b], out_hbm.at[pl.ds(row0, CHUNK)],
                    ssem.at[b]).wait()

            def scale_row(i, _):
                for j in range(D_MODEL // LANES):
                    sbuf[b, i, pl.ds(j * LANES, LANES)] = (
                        gbuf[b, i, pl.ds(j * LANES, LANES)] * SCALE)
                return 0

            lax.fori_loop(0, CHUNK, scale_row, 0, unroll=8)

            @pl.when(g + NBUF < nchunks)
            def _next_gather():
                pltpu.async_copy(
                    lut_hbm.at[idx_v.at[g + NBUF]], gbuf.at[b], gsem.at[b])

            pltpu.async_copy(
                sbuf.at[b], out_hbm.at[pl.ds(row0 + g * CHUNK, CHUNK)],
                ssem.at[b])
        return 0

    lax.fori_loop(0, nouter, outer, 0)

    for b in range(NBUF):
        pltpu.make_async_copy(
            sbuf.at[b], out_hbm.at[pl.ds(row0, CHUNK)], ssem.at[b]).wait()


def kernel(x, lut):
    n_rows, n_cols = x.shape
    total = n_rows * n_cols
    idx = x.reshape(total // CHUNK, CHUNK).astype(jnp.int32)

    emb_flat = pl.kernel(
        _emb_body,
        out_type=jax.ShapeDtypeStruct((total, D_MODEL), jnp.float32),
        mesh=plsc.VectorSubcoreMesh(core_axis_name="c", subcore_axis_name="s"),
        compiler_params=pltpu.CompilerParams(
            use_tc_tiling_on_sc=False,
            skip_device_barrier=True,
            disable_bounds_checks=True,
            disable_semaphore_checks=True,
        ),
        scratch_types=[
            pltpu.VMEM((total // CHUNK // NUM_WORKERS, CHUNK), jnp.int32),
            pltpu.VMEM((NBUF, CHUNK, D_MODEL), jnp.float32),
            pltpu.VMEM((NBUF, CHUNK, D_MODEL), jnp.float32),
            pltpu.SemaphoreType.DMA((NBUF,)),
            pltpu.SemaphoreType.DMA((NBUF,)),
        ],
    )(idx, lut)

    return emb_flat.reshape(n_rows, n_cols, D_MODEL)
